# baseline (speedup 1.0000x reference)
import jax
import jax.numpy as jnp
from jax import lax
from jax.experimental import pallas as pl
from jax.experimental.pallas import tpu as pltpu
from jax.experimental.pallas import tpu_sc as plsc

VOCAB = 1000000
EMB = 32
BATCH = 4096
HIST = 200

NC = 2
NS = 16
NW = NC * NS

BW = BATCH // NW
LANES = 16


def _body(table_hbm, idx_hbm, out_hbm, idx_v, gbuf0, gbuf1, tbuf0, tbuf1,
          gsem0, gsem1, wsem0, wsem1):
    wid = lax.axis_index("s") * NC + lax.axis_index("c")
    b0 = wid * BW
    pltpu.sync_copy(idx_hbm.at[:, pl.ds(b0, BW)], idx_v)

    gbufs = (gbuf0, gbuf1)
    tbufs = (tbuf0, tbuf1)
    gsems = (gsem0, gsem1)
    wsems = (wsem0, wsem1)

    def fire_gather(h, p):
        pltpu.async_copy(table_hbm.at[idx_v.at[h]], gbufs[p], gsems[p])

    def wait_gather(p):
        pltpu.make_async_copy(table_hbm.at[idx_v.at[0]], gbufs[p],
                              gsems[p]).wait()

    def fire_write(h, p):
        for e in range(EMB):
            pltpu.async_copy(tbufs[p].at[pl.ds(e * BW, BW)],
                             out_hbm.at[h, e, pl.ds(b0, BW)], wsems[p])

    def wait_write(p):
        for e in range(EMB):
            pltpu.make_async_copy(tbufs[p].at[pl.ds(e * BW, BW)],
                                  out_hbm.at[0, e, pl.ds(b0, BW)],
                                  wsems[p]).wait()

    def transpose(p):
        g, t = gbufs[p], tbufs[p]
        lane = lax.iota(jnp.int32, LANES)
        lo = lane * BW
        hi = (LANES + lane) * BW
        for j in range(BW):
            plsc.store_scatter(t, [lo + j], g[j, pl.ds(0, LANES)])
            plsc.store_scatter(t, [hi + j], g[j, pl.ds(LANES, LANES)])

    fire_gather(0, 0)
    fire_gather(1, 1)

    def step(i, carry):
        h = 2 * i
        for p in range(2):
            wait_gather(p)

            @pl.when(i > 0)
            def _():
                wait_write(p)

            transpose(p)

            @pl.when(h + p + 2 < HIST)
            def _():
                fire_gather(h + p + 2, p)

            fire_write(h + p, p)
        return carry

    lax.fori_loop(0, HIST // 2, step, 0)
    wait_write(0)
    wait_write(1)


@jax.jit
def _gather(token_table, idx_t):
    mesh = plsc.VectorSubcoreMesh(core_axis_name="c", subcore_axis_name="s")
    f = pl.kernel(
        _body,
        out_type=jax.ShapeDtypeStruct((HIST, EMB, BATCH), jnp.float32),
        mesh=mesh,
        scratch_types=[
            pltpu.VMEM((HIST, BW), jnp.int32),
            pltpu.VMEM((BW, EMB), jnp.float32),
            pltpu.VMEM((BW, EMB), jnp.float32),
            pltpu.VMEM((EMB * BW,), jnp.float32),
            pltpu.VMEM((EMB * BW,), jnp.float32),
            pltpu.SemaphoreType.DMA,
            pltpu.SemaphoreType.DMA,
            pltpu.SemaphoreType.DMA,
            pltpu.SemaphoreType.DMA,
        ],
        compiler_params=pltpu.CompilerParams(use_tc_tiling_on_sc=False,
                                             needs_layout_passes=False),
    )
    return f(token_table, idx_t)


def kernel(node_feats, node_lens, token_table):
    del node_lens
    idx_t = node_feats.T.astype(jnp.int32)
    out_t = _gather(token_table, idx_t)
    return out_t.transpose(2, 0, 1)
